# baseline (speedup 1.0000x reference)
import jax
import jax.numpy as jnp
from jax.experimental import pallas as pl

N_NODES = 10000
DNA_BASE = 0.9
DNA_TEMP = 0.2
RMS_EPS = 1e-8


def _silu(v):
    return v * jax.nn.sigmoid(v)


def _dna_kernel(h_ref, H3_ref, b_ref, out_ref):
    h = h_ref[0]
    H3 = H3_ref[0]
    b = b_ref[0]
    out_ref[...] = (jnp.dot(h, H3, preferred_element_type=jnp.float32) + b)[None]


def _edge_kernel(src_ref, w1_ref, w2_ref, dna_ref, out_ref):
    src = src_ref[...]
    w1 = w1_ref[...]
    w2 = w2_ref[...]
    dna = dna_ref[...]
    S = src.shape[1]
    acc = jnp.zeros_like(src)
    for s in range(S):
        acc += w1[:, s, :, None] * src[:, s, None, :]
    h = _silu(acc)
    acc2 = jnp.zeros_like(src)
    for s in range(S):
        acc2 += w2[:, s, :, None] * h[:, s, None, :]
    gain = 1.0 + DNA_BASE * jnp.tanh(dna[:, :S])
    bias = DNA_TEMP * dna[:, S:]
    out_ref[...] = acc2 * gain[:, :, None] + bias[:, :, None]


def _rms_kernel(st_ref, agg_ref, w_ref, out_ref):
    y = st_ref[...] + agg_ref[...]
    rms = jnp.sqrt(jnp.mean(y * y, axis=1, keepdims=True) + RMS_EPS)
    out_ref[...] = y / rms * w_ref[...]


def kernel(x, W_ctx, b_ctx, H1, Hb1, H2, Hb2, H3, Hb3, dna_scale,
           W_entry, b_entry, mlp_w1, mlp_w2, hop_scale, rms_weight,
           W_out, b_out, edge_src, edge_dst, entry_indices):
    B, _ = x.shape
    SEC, _, HID = H1.shape
    E, S, _ = mlp_w1.shape
    N = N_NODES

    ctx = jnp.mean(x @ W_ctx + b_ctx, axis=0)
    h = _silu(jnp.einsum('c,sch->sh', ctx, H1) + Hb1)
    h = _silu(jnp.einsum('sh,shk->sk', h, H2) + Hb2)

    D = H3.shape[2]
    Db = 3200
    dna_flat = pl.pallas_call(
        _dna_kernel,
        grid=(SEC, D // Db),
        in_specs=[
            pl.BlockSpec((1, 1, HID), lambda s, d: (s, 0, 0)),
            pl.BlockSpec((1, HID, Db), lambda s, d: (s, 0, d)),
            pl.BlockSpec((1, 1, Db), lambda s, d: (s, 0, d)),
        ],
        out_specs=pl.BlockSpec((1, 1, Db), lambda s, d: (s, 0, d)),
        out_shape=jax.ShapeDtypeStruct((SEC, 1, D), jnp.float32),
    )(h.reshape(SEC, 1, HID), H3, Hb3.reshape(SEC, 1, D))
    dna = dna_flat.reshape(-1, 2 * S)[:E] * dna_scale

    entry = (x @ W_entry + b_entry).reshape(B, -1, S)
    statesT = jnp.zeros((N, S, B), x.dtype).at[entry_indices].set(
        jnp.transpose(entry, (1, 2, 0)))

    rw = rms_weight.reshape(1, S, 1)
    Eb = 400
    Nb = 500

    for _ in range(2):
        srcT = statesT[edge_src]
        mT = pl.pallas_call(
            _edge_kernel,
            grid=(E // Eb,),
            in_specs=[
                pl.BlockSpec((Eb, S, B), lambda e: (e, 0, 0)),
                pl.BlockSpec((Eb, S, S), lambda e: (e, 0, 0)),
                pl.BlockSpec((Eb, S, S), lambda e: (e, 0, 0)),
                pl.BlockSpec((Eb, 2 * S), lambda e: (e, 0)),
            ],
            out_specs=pl.BlockSpec((Eb, S, B), lambda e: (e, 0, 0)),
            out_shape=jax.ShapeDtypeStruct((E, S, B), jnp.float32),
        )(srcT, mlp_w1, mlp_w2, dna)
        aggT = jax.ops.segment_sum(mT, edge_dst, num_segments=N) * hop_scale
        statesT = pl.pallas_call(
            _rms_kernel,
            grid=(N // Nb,),
            in_specs=[
                pl.BlockSpec((Nb, S, B), lambda n: (n, 0, 0)),
                pl.BlockSpec((Nb, S, B), lambda n: (n, 0, 0)),
                pl.BlockSpec((1, S, 1), lambda n: (0, 0, 0)),
            ],
            out_specs=pl.BlockSpec((Nb, S, B), lambda n: (n, 0, 0)),
            out_shape=jax.ShapeDtypeStruct((N, S, B), jnp.float32),
        )(statesT, aggT, rw)

    readout = jnp.mean(statesT, axis=0).T
    return readout @ W_out + b_out


# speedup vs baseline: 1.0149x; 1.0149x over previous
import jax
import jax.numpy as jnp
from jax.experimental import pallas as pl

N_NODES = 10000
DNA_BASE = 0.9
DNA_TEMP = 0.2
RMS_EPS = 1e-8


def _silu(v):
    return v * jax.nn.sigmoid(v)


def _dna_kernel(h_ref, H3_ref, b_ref, out_ref):
    h = h_ref[0]
    H3 = H3_ref[0]
    b = b_ref[0]
    out_ref[...] = (jnp.dot(h, H3, preferred_element_type=jnp.float32) + b)[None]


def _mlp_body(src, w1, w2, dna):
    S = src.shape[1]
    acc = jnp.zeros_like(src)
    for s in range(S):
        acc += w1[:, s, :, None] * src[:, s, None, :]
    h = _silu(acc)
    acc2 = jnp.zeros_like(src)
    for s in range(S):
        acc2 += w2[:, s, :, None] * h[:, s, None, :]
    gain = 1.0 + DNA_BASE * jnp.tanh(dna[:, :S])
    bias = DNA_TEMP * dna[:, S:]
    return acc2 * gain[:, :, None] + bias[:, :, None]


def _edge_kernel(src_ref, w1_ref, w2_ref, dna_ref, out_ref):
    out_ref[...] = _mlp_body(src_ref[...], w1_ref[...], w2_ref[...],
                             dna_ref[...])


def _edge_entry_kernel(esrc_ref, entry_ref, w1_ref, w2_ref, dna_ref, out_ref):
    esrc = esrc_ref[...][:, 0]
    entry = entry_ref[...]
    src = jnp.zeros(
        (esrc.shape[0], entry.shape[1], entry.shape[2]), entry.dtype)
    for i in range(entry.shape[0]):
        mask = (esrc == i).astype(entry.dtype)[:, None, None]
        src += mask * entry[i][None]
    out_ref[...] = _mlp_body(src, w1_ref[...], w2_ref[...], dna_ref[...])


def _rms_kernel(y_ref, w_ref, out_ref):
    y = y_ref[...]
    rms = jnp.sqrt(jnp.mean(y * y, axis=1, keepdims=True) + RMS_EPS)
    out_ref[...] = y / rms * w_ref[...]


def kernel(x, W_ctx, b_ctx, H1, Hb1, H2, Hb2, H3, Hb3, dna_scale,
           W_entry, b_entry, mlp_w1, mlp_w2, hop_scale, rms_weight,
           W_out, b_out, edge_src, edge_dst, entry_indices):
    B, _ = x.shape
    SEC, _, HID = H1.shape
    E, S, _ = mlp_w1.shape
    N = N_NODES

    ctx = jnp.mean(x @ W_ctx + b_ctx, axis=0)
    h = _silu(jnp.einsum('c,sch->sh', ctx, H1) + Hb1)
    h = _silu(jnp.einsum('sh,shk->sk', h, H2) + Hb2)

    D = H3.shape[2]
    Db = 3200
    dna_flat = pl.pallas_call(
        _dna_kernel,
        grid=(SEC, D // Db),
        in_specs=[
            pl.BlockSpec((1, 1, HID), lambda s, d: (s, 0, 0)),
            pl.BlockSpec((1, HID, Db), lambda s, d: (s, 0, d)),
            pl.BlockSpec((1, 1, Db), lambda s, d: (s, 0, d)),
        ],
        out_specs=pl.BlockSpec((1, 1, Db), lambda s, d: (s, 0, d)),
        out_shape=jax.ShapeDtypeStruct((SEC, 1, D), jnp.float32),
    )(h.reshape(SEC, 1, HID), H3, Hb3.reshape(SEC, 1, D))
    dna = dna_flat.reshape(-1, 2 * S)[:E] * dna_scale

    entry = (x @ W_entry + b_entry).reshape(B, -1, S)
    entryT = jnp.transpose(entry, (1, 2, 0))
    n_entry = entryT.shape[0]

    rw = rms_weight.reshape(1, S, 1)
    Eb = 400
    Nb = 500
    edge_blocks = [
        pl.BlockSpec((Eb, S, S), lambda e: (e, 0, 0)),
        pl.BlockSpec((Eb, S, S), lambda e: (e, 0, 0)),
        pl.BlockSpec((Eb, 2 * S), lambda e: (e, 0)),
    ]
    m_shape = jax.ShapeDtypeStruct((E, S, B), jnp.float32)
    m_spec = pl.BlockSpec((Eb, S, B), lambda e: (e, 0, 0))

    def rms(y):
        return pl.pallas_call(
            _rms_kernel,
            grid=(N // Nb,),
            in_specs=[
                pl.BlockSpec((Nb, S, B), lambda n: (n, 0, 0)),
                pl.BlockSpec((1, S, 1), lambda n: (0, 0, 0)),
            ],
            out_specs=pl.BlockSpec((Nb, S, B), lambda n: (n, 0, 0)),
            out_shape=jax.ShapeDtypeStruct((N, S, B), jnp.float32),
        )(y, rw)

    mT = pl.pallas_call(
        _edge_entry_kernel,
        grid=(E // Eb,),
        in_specs=[
            pl.BlockSpec((Eb, 1), lambda e: (e, 0)),
            pl.BlockSpec((n_entry, S, B), lambda e: (0, 0, 0)),
        ] + edge_blocks,
        out_specs=m_spec,
        out_shape=m_shape,
    )(edge_src.reshape(E, 1).astype(jnp.int32), entryT, mlp_w1, mlp_w2, dna)
    aggT = jax.ops.segment_sum(mT, edge_dst, num_segments=N) * hop_scale
    statesT = rms(aggT.at[entry_indices].add(entryT))

    srcT = statesT[edge_src]
    mT = pl.pallas_call(
        _edge_kernel,
        grid=(E // Eb,),
        in_specs=[m_spec] + edge_blocks,
        out_specs=m_spec,
        out_shape=m_shape,
    )(srcT, mlp_w1, mlp_w2, dna)
    aggT = jax.ops.segment_sum(mT, edge_dst, num_segments=N) * hop_scale
    statesT = rms(statesT + aggT)

    readout = jnp.mean(statesT, axis=0).T
    return readout @ W_out + b_out
